# baseline (speedup 1.0000x reference)
import math

import jax
import jax.numpy as jnp
import numpy as np
from jax.experimental import pallas as pl
from jax.experimental.pallas import tpu as pltpu

N = 2048
D = 2048
H = 16
DH = 128
K_TOP = 128
NSTAT = 64
INV_SQRT_DH = 1.0 / math.sqrt(DH)

_MASK31 = np.int32(0x7FFFFFFF)


def _f32_keys(s):
    b = jax.lax.bitcast_convert_type(s, jnp.int32)
    return b ^ (jax.lax.shift_right_arithmetic(b, 31) & _MASK31)


def _keys_f32(k):
    b = k ^ (jax.lax.shift_right_arithmetic(k, 31) & _MASK31)
    return jax.lax.bitcast_convert_type(b, jnp.float32)


def _proj_body(x_ref, w_ref, b_ref, o_ref):
    o_ref[...] = (
        jax.lax.dot_general(
            x_ref[...], w_ref[...], (((1,), (1,)), ((), ())),
            preferred_element_type=jnp.float32,
        )
        + b_ref[...]
    )


def _row_threshold(s):
    keys = _f32_keys(s)
    row_max = jnp.max(s, axis=1, keepdims=True)

    h16 = jax.lax.shift_right_arithmetic(keys, 16).astype(jnp.int16)
    l16 = (keys ^ np.int32(0x8000)).astype(jnp.int16)

    b_lo0 = jnp.full((s.shape[0], 1), -32768, jnp.int32)
    b_hi0 = jnp.full((s.shape[0], 1), 32768, jnp.int32)

    def body1(_, st):
        lo, hi = st
        mid = jax.lax.shift_right_arithmetic(lo + hi, 1)
        cnt = jnp.sum(
            h16 >= mid.astype(jnp.int16), axis=1, keepdims=True,
            dtype=jnp.int32,
        )
        ge = cnt >= K_TOP
        return jnp.where(ge, mid, lo), jnp.where(ge, hi, mid)

    b, _ = jax.lax.fori_loop(0, 16, body1, (b_lo0, b_hi0))

    b16 = b.astype(jnp.int16)
    c_gt = jnp.sum(h16 > b16, axis=1, keepdims=True, dtype=jnp.int32)
    r = K_TOP - c_gt
    eq = h16 == b16

    u_lo0 = jnp.full_like(b, -32768)
    u_hi0 = jnp.full_like(b, 32768)

    def body2(_, st):
        lo, hi = st
        mid = jax.lax.shift_right_arithmetic(lo + hi, 1)
        cnt = jnp.sum(
            jnp.logical_and(eq, l16 >= mid.astype(jnp.int16)),
            axis=1, keepdims=True, dtype=jnp.int32,
        )
        ge = cnt >= r
        return jnp.where(ge, mid, lo), jnp.where(ge, hi, mid)

    u, _ = jax.lax.fori_loop(0, 16, body2, (u_lo0, u_hi0))

    t_key = jax.lax.shift_left(b, 16) | (
        (u & np.int32(0xFFFF)) ^ np.int32(0x8000)
    )
    return _keys_f32(t_key), row_max


def _fused_attn_body(q_ref, k_ref, v_ref, o_ref):
    q = q_ref[...]
    s = jax.lax.dot_general(
        q, k_ref[...], (((1,), (1,)), ((), ())),
        preferred_element_type=jnp.float32,
    ) * INV_SQRT_DH

    t, m = _row_threshold(s)

    p = jnp.where(s >= t, jnp.exp(s - m), 0.0)
    z = jnp.sum(p, axis=1, keepdims=True)
    o = jax.lax.dot_general(
        p, v_ref[...], (((1,), (0,)), ((), ())),
        preferred_element_type=jnp.float32,
    )
    o_ref[...] = o / z


def kernel(x, Wq, bq, Wk, bk, Wv, bv, Wo, bo, static_keys):
    x2 = x[0]

    RB = 256
    CB = 512

    def proj(w, b):
        return pl.pallas_call(
            _proj_body,
            grid=(N // RB, D // CB),
            in_specs=[
                pl.BlockSpec((RB, D), lambda r, c: (r, 0)),
                pl.BlockSpec((CB, D), lambda r, c: (c, 0)),
                pl.BlockSpec((1, CB), lambda r, c: (0, c)),
            ],
            out_specs=pl.BlockSpec((RB, CB), lambda r, c: (r, c)),
            out_shape=jax.ShapeDtypeStruct((N, D), jnp.float32),
        )(x2, w, b[None, :])

    q = proj(Wq, bq)
    k_dyn = proj(Wk, bk)
    v = proj(Wv, bv)

    k_full = jnp.concatenate([static_keys, k_dyn[NSTAT:]], axis=0)

    attn_out = pl.pallas_call(
        _fused_attn_body,
        grid=(H, N // RB),
        in_specs=[
            pl.BlockSpec((RB, DH), lambda h, r: (r, h)),
            pl.BlockSpec((N, DH), lambda h, r: (0, h)),
            pl.BlockSpec((N, DH), lambda h, r: (0, h)),
        ],
        out_specs=pl.BlockSpec((RB, DH), lambda h, r: (r, h)),
        out_shape=jax.ShapeDtypeStruct((N, D), jnp.float32),
    )(q, k_full, v)

    final = pl.pallas_call(
        _proj_body,
        grid=(N // RB, D // CB),
        in_specs=[
            pl.BlockSpec((RB, D), lambda r, c: (r, 0)),
            pl.BlockSpec((CB, D), lambda r, c: (c, 0)),
            pl.BlockSpec((1, CB), lambda r, c: (0, c)),
        ],
        out_specs=pl.BlockSpec((RB, CB), lambda r, c: (r, c)),
        out_shape=jax.ShapeDtypeStruct((N, D), jnp.float32),
    )(attn_out, Wo, bo[None, :])

    return final[None]


# speedup vs baseline: 1.7172x; 1.7172x over previous
import math

import jax
import jax.numpy as jnp
import numpy as np
from jax.experimental import pallas as pl
from jax.experimental.pallas import tpu as pltpu

N = 2048
D = 2048
H = 16
DH = 128
K_TOP = 128
NSTAT = 64
INV_SQRT_DH = 1.0 / math.sqrt(DH)

_MASK31 = np.int32(0x7FFFFFFF)


def _f32_keys(s):
    b = jax.lax.bitcast_convert_type(s, jnp.int32)
    return b ^ (jax.lax.shift_right_arithmetic(b, 31) & _MASK31)


def _keys_f32(k):
    b = k ^ (jax.lax.shift_right_arithmetic(k, 31) & _MASK31)
    return jax.lax.bitcast_convert_type(b, jnp.float32)


def _proj_body(x_ref, w_ref, b_ref, o_ref):
    o_ref[...] = (
        jax.lax.dot_general(
            x_ref[...], w_ref[...], (((1,), (1,)), ((), ())),
            preferred_element_type=jnp.float32,
        )
        + b_ref[...]
    )


def _row_threshold(s):
    keys = _f32_keys(s)
    row_max = jnp.max(s, axis=1, keepdims=True)
    lo0 = jnp.min(keys, axis=1, keepdims=True)
    hi0 = jnp.max(keys, axis=1, keepdims=True) + np.int32(1)

    def body(_, st):
        lo, hi = st
        mid = lo + jax.lax.shift_right_logical(hi - lo, 1)
        cnt = jnp.sum(keys >= mid, axis=1, keepdims=True, dtype=jnp.int32)
        ge = cnt >= K_TOP
        return jnp.where(ge, mid, lo), jnp.where(ge, hi, mid)

    lo, _ = jax.lax.fori_loop(0, 32, body, (lo0, hi0))
    return _keys_f32(lo), row_max


def _fused_attn_body(q_ref, k_ref, v_ref, o_ref):
    q = q_ref[...]
    s = jax.lax.dot_general(
        q, k_ref[...], (((1,), (1,)), ((), ())),
        preferred_element_type=jnp.float32,
    ) * INV_SQRT_DH

    t, m = _row_threshold(s)

    p = jnp.where(s >= t, jnp.exp(s - m), 0.0)
    z = jnp.sum(p, axis=1, keepdims=True)
    o = jax.lax.dot_general(
        p, v_ref[...], (((1,), (0,)), ((), ())),
        preferred_element_type=jnp.float32,
    )
    o_ref[...] = o / z


def kernel(x, Wq, bq, Wk, bk, Wv, bv, Wo, bo, static_keys):
    x2 = x[0]

    RB = 256
    AB = 512
    CB = 512

    def proj(w, b):
        return pl.pallas_call(
            _proj_body,
            grid=(N // RB, D // CB),
            in_specs=[
                pl.BlockSpec((RB, D), lambda r, c: (r, 0)),
                pl.BlockSpec((CB, D), lambda r, c: (c, 0)),
                pl.BlockSpec((1, CB), lambda r, c: (0, c)),
            ],
            out_specs=pl.BlockSpec((RB, CB), lambda r, c: (r, c)),
            out_shape=jax.ShapeDtypeStruct((N, D), jnp.float32),
        )(x2, w, b[None, :])

    q = proj(Wq, bq)
    k_dyn = proj(Wk, bk)
    v = proj(Wv, bv)

    k_full = jnp.concatenate([static_keys, k_dyn[NSTAT:]], axis=0)

    attn_out = pl.pallas_call(
        _fused_attn_body,
        grid=(H, N // AB),
        in_specs=[
            pl.BlockSpec((AB, DH), lambda h, r: (r, h)),
            pl.BlockSpec((N, DH), lambda h, r: (0, h)),
            pl.BlockSpec((N, DH), lambda h, r: (0, h)),
        ],
        out_specs=pl.BlockSpec((AB, DH), lambda h, r: (r, h)),
        out_shape=jax.ShapeDtypeStruct((N, D), jnp.float32),
    )(q, k_full, v)

    final = pl.pallas_call(
        _proj_body,
        grid=(N // RB, D // CB),
        in_specs=[
            pl.BlockSpec((RB, D), lambda r, c: (r, 0)),
            pl.BlockSpec((CB, D), lambda r, c: (c, 0)),
            pl.BlockSpec((1, CB), lambda r, c: (0, c)),
        ],
        out_specs=pl.BlockSpec((RB, CB), lambda r, c: (r, c)),
        out_shape=jax.ShapeDtypeStruct((N, D), jnp.float32),
    )(attn_out, Wo, bo[None, :])

    return final[None]
